# baseline (speedup 1.0000x reference)
import functools

import jax
import jax.numpy as jnp
from jax import lax
from jax.experimental import pallas as pl
from jax.experimental.pallas import tpu as pltpu
from jax.experimental.pallas import tpu_sc as plsc

N = 10000
E = 320000
D = 128
NC = 2
NS = 16
NW = NC * NS
EPW = 10240
EPAD = EPW * NW
CHUNK = 64
NCHUNK = EPW // CHUNK
NBUF = 2
NG = NCHUNK // NBUF
NPAD = 10240
RPT = NPAD // NS

_sc_mesh = plsc.VectorSubcoreMesh(core_axis_name="c", subcore_axis_name="s")


@functools.partial(
    pl.kernel,
    out_type=jax.ShapeDtypeStruct((NC, NPAD, D), jnp.float32),
    mesh=_sc_mesh,
    scratch_types=[
        pltpu.VMEM((NCHUNK, CHUNK), jnp.int32),
        pltpu.VMEM((CHUNK, D), jnp.float32),
        pltpu.VMEM_SHARED((NPAD, D), jnp.float32),
    ] + [pltpu.SemaphoreType.DMA] * NBUF,
)
def _deg_kernel(col_hbm, ones_hbm, zeros_hbm, out_hbm, idx_c, ones_v, tab,
                *sems):
    c = lax.axis_index("c")
    s = lax.axis_index("s")
    wid = c * NS + s
    pltpu.sync_copy(zeros_hbm, tab.at[pl.ds(s * RPT, RPT)])
    pltpu.sync_copy(col_hbm.at[wid], idx_c)
    pltpu.sync_copy(ones_hbm, ones_v)
    plsc.subcore_barrier()

    def body(g, carry):
        for b in range(NBUF):
            i = g * NBUF + b

            @pl.when(g > 0)
            def _():
                pltpu.make_async_copy(
                    ones_v, tab.at[idx_c.at[0]], sems[b]).wait()

            pltpu.async_copy(ones_v, tab.at[idx_c.at[i]], sems[b], add=True)
        return carry

    lax.fori_loop(0, NG, body, 0)
    for b in range(NBUF):
        pltpu.make_async_copy(ones_v, tab.at[idx_c.at[0]], sems[b]).wait()
    plsc.subcore_barrier()
    pltpu.sync_copy(tab.at[pl.ds(s * RPT, RPT)],
                    out_hbm.at[c, pl.ds(s * RPT, RPT)])


@functools.partial(
    pl.kernel,
    out_type=jax.ShapeDtypeStruct((NC, NPAD, D), jnp.float32),
    mesh=_sc_mesh,
    scratch_types=[
        pltpu.VMEM((EPW,), jnp.int32),
        pltpu.VMEM((NCHUNK, CHUNK), jnp.int32),
    ] + [pltpu.VMEM((CHUNK, D), jnp.float32)] * NBUF + [
        pltpu.VMEM_SHARED((NPAD, D), jnp.float32),
    ] + [pltpu.SemaphoreType.DMA] * (2 * NBUF),
)
def _hop_kernel(u_hbm, row_hbm, col_hbm, zeros_hbm, out_hbm,
                idx_r, idx_c, *bufs):
    rows = bufs[:NBUF]
    acc = bufs[NBUF]
    semg = bufs[NBUF + 1:NBUF + 1 + NBUF]
    sems_ = bufs[NBUF + 1 + NBUF:]
    c = lax.axis_index("c")
    s = lax.axis_index("s")
    wid = c * NS + s
    pltpu.sync_copy(zeros_hbm, acc.at[pl.ds(s * RPT, RPT)])
    pltpu.sync_copy(row_hbm.at[wid], idx_r)
    pltpu.sync_copy(col_hbm.at[wid], idx_c)
    plsc.subcore_barrier()

    def body(g, carry):
        for b in range(NBUF):
            i = g * NBUF + b

            @pl.when(g > 0)
            def _():
                pltpu.make_async_copy(
                    rows[b], acc.at[idx_c.at[0]], sems_[b]).wait()

            off = pl.multiple_of(i * CHUNK, 8)
            pltpu.async_copy(u_hbm.at[idx_r.at[pl.ds(off, CHUNK)]],
                             rows[b], semg[b])
        for b in range(NBUF):
            i = g * NBUF + b
            pltpu.make_async_copy(
                u_hbm.at[idx_r.at[pl.ds(0, CHUNK)]], rows[b], semg[b]).wait()
            pltpu.async_copy(rows[b], acc.at[idx_c.at[i]], sems_[b],
                             add=True)
        return carry

    lax.fori_loop(0, NG, body, 0)
    for b in range(NBUF):
        pltpu.make_async_copy(rows[b], acc.at[idx_c.at[0]], sems_[b]).wait()
    plsc.subcore_barrier()
    pltpu.sync_copy(acc.at[pl.ds(s * RPT, RPT)],
                    out_hbm.at[c, pl.ds(s * RPT, RPT)])


_R = 2000
_GRID = N // _R


def _deg_of(d0, d1):
    return d0[:, :1] + d1[:, :1] + 1.0


def _mm_body(x_ref, wt_ref, d0_ref, d1_ref, y_ref):
    dinv = lax.rsqrt(_deg_of(d0_ref[...], d1_ref[...]))
    y_ref[...] = dinv * jnp.dot(x_ref[...], wt_ref[...],
                                preferred_element_type=jnp.float32)


def _comb_body(p0_ref, p1_ref, u_ref, d0_ref, d1_ref, o_ref):
    deg = _deg_of(d0_ref[...], d1_ref[...])
    o_ref[...] = (p0_ref[...] + p1_ref[...] + u_ref[...]) / deg


def _final_body(p0_ref, p1_ref, u_ref, d0_ref, d1_ref, b_ref, o_ref):
    dinv = lax.rsqrt(_deg_of(d0_ref[...], d1_ref[...]))
    o_ref[...] = (p0_ref[...] + p1_ref[...] + u_ref[...]) * dinv + b_ref[...]


_row_spec = pl.BlockSpec((_R, D), lambda i: (i, 0))
_w_spec = pl.BlockSpec((D, D), lambda i: (0, 0))
_b_spec = pl.BlockSpec((1, D), lambda i: (0, 0))
_out_row = jax.ShapeDtypeStruct((N, D), jnp.float32)

_mm_call = pl.pallas_call(
    _mm_body, grid=(_GRID,),
    in_specs=[_row_spec, _w_spec, _row_spec, _row_spec],
    out_specs=_row_spec, out_shape=_out_row)

_comb_call = pl.pallas_call(
    _comb_body, grid=(_GRID,),
    in_specs=[_row_spec, _row_spec, _row_spec, _row_spec, _row_spec],
    out_specs=_row_spec, out_shape=_out_row)

_final_call = pl.pallas_call(
    _final_body, grid=(_GRID,),
    in_specs=[_row_spec, _row_spec, _row_spec, _row_spec, _row_spec, _b_spec],
    out_specs=_row_spec, out_shape=_out_row)


def kernel(x, edge_index, W, b):
    npad = EPAD - E
    row = jnp.concatenate([edge_index[0], jnp.zeros((npad,), jnp.int32)])
    col = jnp.concatenate([edge_index[1], jnp.full((npad,), N, jnp.int32)])
    row = row.reshape(NW, EPW)
    col = col.reshape(NW, NCHUNK, CHUNK)
    wt = W.T
    ones_deg = jnp.ones((CHUNK, D), jnp.float32)
    zeros_row = jnp.zeros((RPT, D), jnp.float32)
    b2 = b.reshape(1, D)

    degp = _deg_kernel(col, ones_deg, zeros_row)
    d0, d1 = degp[0], degp[1]

    u0 = _mm_call(x, wt, d0, d1)
    p = _hop_kernel(u0, row, col, zeros_row)
    u1 = _comb_call(p[0], p[1], u0, d0, d1)
    p = _hop_kernel(u1, row, col, zeros_row)
    out = _final_call(p[0], p[1], u1, d0, d1, b2)
    return (out, out)


# speedup vs baseline: 2.5468x; 2.5468x over previous
import functools

import jax
import jax.numpy as jnp
from jax import lax
from jax.experimental import pallas as pl
from jax.experimental.pallas import tpu as pltpu
from jax.experimental.pallas import tpu_sc as plsc

N = 10000
E = 320000
D = 128
NC = 2
NS = 16
NW = NC * NS
EPW = 10240
EPAD = EPW * NW
CHUNK = 64
NCHUNK = EPW // CHUNK
NBUF = 2
NG = NCHUNK // NBUF
NPAD = 10240
RPT = NPAD // NS

_sc_mesh = plsc.VectorSubcoreMesh(core_axis_name="c", subcore_axis_name="s")


@functools.partial(
    pl.kernel,
    out_type=jax.ShapeDtypeStruct((NC, NPAD, D), jnp.float32),
    mesh=_sc_mesh,
    scratch_types=[
        pltpu.VMEM((NCHUNK, CHUNK), jnp.int32),
        pltpu.VMEM((CHUNK, D), jnp.float32),
        pltpu.VMEM_SHARED((NPAD, D), jnp.float32),
    ] + [pltpu.SemaphoreType.DMA] * NBUF,
)
def _deg_kernel(col_hbm, ones_hbm, zeros_hbm, out_hbm, idx_c, ones_v, tab,
                *sems):
    c = lax.axis_index("c")
    s = lax.axis_index("s")
    wid = c * NS + s
    pltpu.sync_copy(zeros_hbm, tab.at[pl.ds(s * RPT, RPT)])
    pltpu.sync_copy(col_hbm.at[wid], idx_c)
    pltpu.sync_copy(ones_hbm, ones_v)
    plsc.subcore_barrier()

    def body(g, carry):
        for b in range(NBUF):
            i = g * NBUF + b

            @pl.when(g > 0)
            def _():
                pltpu.make_async_copy(
                    ones_v, tab.at[idx_c.at[0]], sems[b]).wait()

            pltpu.async_copy(ones_v, tab.at[idx_c.at[i]], sems[b], add=True)
        return carry

    lax.fori_loop(0, NG, body, 0)
    for b in range(NBUF):
        pltpu.make_async_copy(ones_v, tab.at[idx_c.at[0]], sems[b]).wait()
    plsc.subcore_barrier()
    pltpu.sync_copy(tab.at[pl.ds(s * RPT, RPT)],
                    out_hbm.at[c, pl.ds(s * RPT, RPT)])


@functools.partial(
    pl.kernel,
    out_type=jax.ShapeDtypeStruct((NC, NPAD, D), jnp.float32),
    mesh=_sc_mesh,
    scratch_types=[
        pltpu.VMEM((EPW,), jnp.int32),
        pltpu.VMEM((NCHUNK, CHUNK), jnp.int32),
    ] + [pltpu.VMEM((CHUNK, D), jnp.float32)] * NBUF + [
        pltpu.VMEM_SHARED((NPAD, D), jnp.float32),
    ] + [pltpu.SemaphoreType.DMA] * (2 * NBUF),
)
def _hop_kernel(u_hbm, row_hbm, col_hbm, zeros_hbm, out_hbm,
                idx_r, idx_c, *bufs):
    rows = bufs[:NBUF]
    acc = bufs[NBUF]
    semg = bufs[NBUF + 1:NBUF + 1 + NBUF]
    sems_ = bufs[NBUF + 1 + NBUF:]
    c = lax.axis_index("c")
    s = lax.axis_index("s")
    wid = c * NS + s
    pltpu.sync_copy(zeros_hbm, acc.at[pl.ds(s * RPT, RPT)])
    pltpu.sync_copy(row_hbm.at[wid], idx_r)
    pltpu.sync_copy(col_hbm.at[wid], idx_c)
    plsc.subcore_barrier()

    def body(g, carry):
        for b in range(NBUF):
            i = g * NBUF + b

            @pl.when(g > 0)
            def _():
                pltpu.make_async_copy(
                    rows[b], acc.at[idx_c.at[0]], sems_[b]).wait()

            off = pl.multiple_of(i * CHUNK, 8)
            pltpu.async_copy(u_hbm.at[idx_r.at[pl.ds(off, CHUNK)]],
                             rows[b], semg[b])
        for b in range(NBUF):
            i = g * NBUF + b
            pltpu.make_async_copy(
                u_hbm.at[idx_r.at[pl.ds(0, CHUNK)]], rows[b], semg[b]).wait()
            pltpu.async_copy(rows[b], acc.at[idx_c.at[i]], sems_[b],
                             add=True)
        return carry

    lax.fori_loop(0, NG, body, 0)
    for b in range(NBUF):
        pltpu.make_async_copy(rows[b], acc.at[idx_c.at[0]], sems_[b]).wait()
    plsc.subcore_barrier()
    pltpu.sync_copy(acc.at[pl.ds(s * RPT, RPT)],
                    out_hbm.at[c, pl.ds(s * RPT, RPT)])


_R = 2000
_GRID = N // _R


def _deg_of(d0, d1):
    return d0[:, :1] + d1[:, :1] + 1.0


def _mm_body(x_ref, wt_ref, d0_ref, d1_ref, y_ref):
    dinv = lax.rsqrt(_deg_of(d0_ref[...], d1_ref[...]))
    y_ref[...] = dinv * jnp.dot(x_ref[...], wt_ref[...],
                                preferred_element_type=jnp.float32)


def _comb_body(p0_ref, p1_ref, u_ref, d0_ref, d1_ref, o_ref):
    deg = _deg_of(d0_ref[...], d1_ref[...])
    o_ref[...] = (p0_ref[...] + p1_ref[...] + u_ref[...]) / deg


def _final_body(p0_ref, p1_ref, u_ref, d0_ref, d1_ref, b_ref, o_ref):
    dinv = lax.rsqrt(_deg_of(d0_ref[...], d1_ref[...]))
    o_ref[...] = (p0_ref[...] + p1_ref[...] + u_ref[...]) * dinv + b_ref[...]


_row_spec = pl.BlockSpec((_R, D), lambda i: (i, 0))
_w_spec = pl.BlockSpec((D, D), lambda i: (0, 0))
_b_spec = pl.BlockSpec((1, D), lambda i: (0, 0))
_out_row = jax.ShapeDtypeStruct((N, D), jnp.float32)

_mm_call = pl.pallas_call(
    _mm_body, grid=(_GRID,),
    in_specs=[_row_spec, _w_spec, _row_spec, _row_spec],
    out_specs=_row_spec, out_shape=_out_row)

_comb_call = pl.pallas_call(
    _comb_body, grid=(_GRID,),
    in_specs=[_row_spec, _row_spec, _row_spec, _row_spec, _row_spec],
    out_specs=_row_spec, out_shape=_out_row)

_final_call = pl.pallas_call(
    _final_body, grid=(_GRID,),
    in_specs=[_row_spec, _row_spec, _row_spec, _row_spec, _row_spec, _b_spec],
    out_specs=_row_spec, out_shape=_out_row)


def kernel(x, edge_index, W, b):
    npad = EPAD - E
    ar = jnp.arange(npad, dtype=jnp.int32)
    row = jnp.concatenate([edge_index[0], ar % N])
    col = jnp.concatenate([edge_index[1], N + ar % (NPAD - N)])
    row = row.reshape(NW, EPW)
    col = col.reshape(NW, NCHUNK, CHUNK)
    wt = W.T
    ones_deg = jnp.ones((CHUNK, D), jnp.float32)
    zeros_row = jnp.zeros((RPT, D), jnp.float32)
    b2 = b.reshape(1, D)

    degp = _deg_kernel(col, ones_deg, zeros_row)
    d0, d1 = degp[0], degp[1]

    u0 = _mm_call(x, wt, d0, d1)
    p = _hop_kernel(u0, row, col, zeros_row)
    u1 = _comb_call(p[0], p[1], u0, d0, d1)
    p = _hop_kernel(u1, row, col, zeros_row)
    out = _final_call(p[0], p[1], u1, d0, d1, b2)
    return (out, out)


# speedup vs baseline: 2.7744x; 1.0893x over previous
import functools

import jax
import jax.numpy as jnp
from jax import lax
from jax.experimental import pallas as pl
from jax.experimental.pallas import tpu as pltpu
from jax.experimental.pallas import tpu_sc as plsc

N = 10000
E = 320000
D = 128
NC = 2
NS = 16
NW = NC * NS
EPW = 10240
EPAD = EPW * NW
CHUNK = 128
NCHUNK = EPW // CHUNK
NBUF = 2
NG = NCHUNK // NBUF
NPAD = 10240
RPT = NPAD // NS

_sc_mesh = plsc.VectorSubcoreMesh(core_axis_name="c", subcore_axis_name="s")


@functools.partial(
    pl.kernel,
    out_type=jax.ShapeDtypeStruct((NC, NPAD, D), jnp.float32),
    mesh=_sc_mesh,
    scratch_types=[
        pltpu.VMEM((NCHUNK, CHUNK), jnp.int32),
        pltpu.VMEM((CHUNK, D), jnp.float32),
        pltpu.VMEM_SHARED((NPAD, D), jnp.float32),
    ] + [pltpu.SemaphoreType.DMA] * NBUF,
)
def _deg_kernel(col_hbm, ones_hbm, zeros_hbm, out_hbm, idx_c, ones_v, tab,
                *sems):
    c = lax.axis_index("c")
    s = lax.axis_index("s")
    wid = c * NS + s
    pltpu.sync_copy(zeros_hbm, tab.at[pl.ds(s * RPT, RPT)])
    pltpu.sync_copy(col_hbm.at[wid], idx_c)
    pltpu.sync_copy(ones_hbm, ones_v)
    plsc.subcore_barrier()

    def body(g, carry):
        for b in range(NBUF):
            i = g * NBUF + b

            @pl.when(g > 0)
            def _():
                pltpu.make_async_copy(
                    ones_v, tab.at[idx_c.at[0]], sems[b]).wait()

            pltpu.async_copy(ones_v, tab.at[idx_c.at[i]], sems[b], add=True)
        return carry

    lax.fori_loop(0, NG, body, 0)
    for b in range(NBUF):
        pltpu.make_async_copy(ones_v, tab.at[idx_c.at[0]], sems[b]).wait()
    plsc.subcore_barrier()
    pltpu.sync_copy(tab.at[pl.ds(s * RPT, RPT)],
                    out_hbm.at[c, pl.ds(s * RPT, RPT)])


@functools.partial(
    pl.kernel,
    out_type=jax.ShapeDtypeStruct((NC, NPAD, D), jnp.float32),
    mesh=_sc_mesh,
    scratch_types=[
        pltpu.VMEM((NCHUNK, CHUNK), jnp.int32),
    ] + [pltpu.VMEM((CHUNK,), jnp.int32)] * NBUF
      + [pltpu.VMEM((CHUNK, D), jnp.float32)] * NBUF + [
        pltpu.VMEM_SHARED((NPAD, D), jnp.float32),
    ] + [pltpu.SemaphoreType.DMA] * (3 * NBUF),
)
def _hop_kernel(u_hbm, row_hbm, col_hbm, zeros_hbm, out_hbm,
                idx_c, *bufs):
    idxb = bufs[:NBUF]
    rows = bufs[NBUF:2 * NBUF]
    acc = bufs[2 * NBUF]
    semi = bufs[2 * NBUF + 1:2 * NBUF + 1 + NBUF]
    semg = bufs[2 * NBUF + 1 + NBUF:2 * NBUF + 1 + 2 * NBUF]
    sems_ = bufs[2 * NBUF + 1 + 2 * NBUF:]
    c = lax.axis_index("c")
    s = lax.axis_index("s")
    wid = c * NS + s
    pltpu.sync_copy(zeros_hbm, acc.at[pl.ds(s * RPT, RPT)])
    pltpu.sync_copy(col_hbm.at[wid], idx_c)
    for b in range(NBUF):
        pltpu.async_copy(row_hbm.at[wid, b], idxb[b], semi[b])
    plsc.subcore_barrier()

    def body(g, carry):
        for b in range(NBUF):
            i = g * NBUF + b

            @pl.when(g > 0)
            def _():
                pltpu.make_async_copy(
                    rows[b], acc.at[idx_c.at[0]], sems_[b]).wait()

            pltpu.make_async_copy(row_hbm.at[wid, 0], idxb[b], semi[b]).wait()
            pltpu.async_copy(u_hbm.at[idxb[b]], rows[b], semg[b])
        for b in range(NBUF):
            i = g * NBUF + b
            pltpu.make_async_copy(
                u_hbm.at[idxb[b]], rows[b], semg[b]).wait()

            @pl.when(g < NG - 1)
            def _():
                pltpu.async_copy(row_hbm.at[wid, i + NBUF], idxb[b], semi[b])

            pltpu.async_copy(rows[b], acc.at[idx_c.at[i]], sems_[b],
                             add=True)
        return carry

    lax.fori_loop(0, NG, body, 0)
    for b in range(NBUF):
        pltpu.make_async_copy(rows[b], acc.at[idx_c.at[0]], sems_[b]).wait()
    plsc.subcore_barrier()
    pltpu.sync_copy(acc.at[pl.ds(s * RPT, RPT)],
                    out_hbm.at[c, pl.ds(s * RPT, RPT)])


_R = 2000
_GRID = N // _R


def _deg_of(d0, d1):
    return d0[:, :1] + d1[:, :1] + 1.0


def _mm_body(x_ref, wt_ref, d0_ref, d1_ref, y_ref):
    dinv = lax.rsqrt(_deg_of(d0_ref[...], d1_ref[...]))
    y_ref[...] = dinv * jnp.dot(x_ref[...], wt_ref[...],
                                preferred_element_type=jnp.float32)


def _comb_body(p0_ref, p1_ref, u_ref, d0_ref, d1_ref, o_ref):
    deg = _deg_of(d0_ref[...], d1_ref[...])
    o_ref[...] = (p0_ref[...] + p1_ref[...] + u_ref[...]) / deg


def _final_body(p0_ref, p1_ref, u_ref, d0_ref, d1_ref, b_ref, o_ref):
    dinv = lax.rsqrt(_deg_of(d0_ref[...], d1_ref[...]))
    o_ref[...] = (p0_ref[...] + p1_ref[...] + u_ref[...]) * dinv + b_ref[...]


_row_spec = pl.BlockSpec((_R, D), lambda i: (i, 0))
_w_spec = pl.BlockSpec((D, D), lambda i: (0, 0))
_b_spec = pl.BlockSpec((1, D), lambda i: (0, 0))
_out_row = jax.ShapeDtypeStruct((N, D), jnp.float32)

_mm_call = pl.pallas_call(
    _mm_body, grid=(_GRID,),
    in_specs=[_row_spec, _w_spec, _row_spec, _row_spec],
    out_specs=_row_spec, out_shape=_out_row)

_comb_call = pl.pallas_call(
    _comb_body, grid=(_GRID,),
    in_specs=[_row_spec, _row_spec, _row_spec, _row_spec, _row_spec],
    out_specs=_row_spec, out_shape=_out_row)

_final_call = pl.pallas_call(
    _final_body, grid=(_GRID,),
    in_specs=[_row_spec, _row_spec, _row_spec, _row_spec, _row_spec, _b_spec],
    out_specs=_row_spec, out_shape=_out_row)


def kernel(x, edge_index, W, b):
    npad = EPAD - E
    ar = jnp.arange(npad, dtype=jnp.int32)
    row = jnp.concatenate([edge_index[0], ar % N])
    col = jnp.concatenate([edge_index[1], N + ar % (NPAD - N)])
    row = row.reshape(NW, NCHUNK, CHUNK)
    col = col.reshape(NW, NCHUNK, CHUNK)
    wt = W.T
    ones_deg = jnp.ones((CHUNK, D), jnp.float32)
    zeros_row = jnp.zeros((RPT, D), jnp.float32)
    b2 = b.reshape(1, D)

    degp = _deg_kernel(col, ones_deg, zeros_row)
    d0, d1 = degp[0], degp[1]

    u0 = _mm_call(x, wt, d0, d1)
    p = _hop_kernel(u0, row, col, zeros_row)
    u1 = _comb_call(p[0], p[1], u0, d0, d1)
    p = _hop_kernel(u1, row, col, zeros_row)
    out = _final_call(p[0], p[1], u1, d0, d1, b2)
    return (out, out)


# speedup vs baseline: 3.0499x; 1.0993x over previous
import functools

import jax
import jax.numpy as jnp
from jax import lax
from jax.experimental import pallas as pl
from jax.experimental.pallas import tpu as pltpu
from jax.experimental.pallas import tpu_sc as plsc

N = 10000
E = 320000
D = 128
NC = 2
NS = 16
NW = NC * NS
EPW = 10240
EPAD = EPW * NW
CHUNK = 128
NCHUNK = EPW // CHUNK
NBUF = 2
NG = NCHUNK // NBUF
HCH = 80
HNCH = EPW // HCH
HNB = 3
HNG = HNCH // HNB
HREM = HNCH - HNG * HNB
NPAD = 10240
RPT = NPAD // NS

_sc_mesh = plsc.VectorSubcoreMesh(core_axis_name="c", subcore_axis_name="s")


@functools.partial(
    pl.kernel,
    out_type=jax.ShapeDtypeStruct((NC, NPAD, D), jnp.float32),
    mesh=_sc_mesh,
    scratch_types=[
        pltpu.VMEM((NCHUNK, CHUNK), jnp.int32),
        pltpu.VMEM((CHUNK, D), jnp.float32),
        pltpu.VMEM_SHARED((NPAD, D), jnp.float32),
    ] + [pltpu.SemaphoreType.DMA] * NBUF,
)
def _deg_kernel(col_hbm, ones_hbm, zeros_hbm, out_hbm, idx_c, ones_v, tab,
                *sems):
    c = lax.axis_index("c")
    s = lax.axis_index("s")
    wid = c * NS + s
    pltpu.sync_copy(zeros_hbm, tab.at[pl.ds(s * RPT, RPT)])
    pltpu.sync_copy(col_hbm.at[wid], idx_c)
    pltpu.sync_copy(ones_hbm, ones_v)
    plsc.subcore_barrier()

    def body(g, carry):
        for b in range(NBUF):
            i = g * NBUF + b

            @pl.when(g > 0)
            def _():
                pltpu.make_async_copy(
                    ones_v, tab.at[idx_c.at[0]], sems[b]).wait()

            pltpu.async_copy(ones_v, tab.at[idx_c.at[i]], sems[b], add=True)
        return carry

    lax.fori_loop(0, NG, body, 0)
    for b in range(NBUF):
        pltpu.make_async_copy(ones_v, tab.at[idx_c.at[0]], sems[b]).wait()
    plsc.subcore_barrier()
    pltpu.sync_copy(tab.at[pl.ds(s * RPT, RPT)],
                    out_hbm.at[c, pl.ds(s * RPT, RPT)])


@functools.partial(
    pl.kernel,
    out_type=jax.ShapeDtypeStruct((NC, NPAD, D), jnp.float32),
    mesh=_sc_mesh,
    scratch_types=[
        pltpu.VMEM((HNCH, HCH), jnp.int32),
    ] + [pltpu.VMEM((HCH,), jnp.int32)] * HNB
      + [pltpu.VMEM((HCH, D), jnp.float32)] * HNB + [
        pltpu.VMEM_SHARED((NPAD, D), jnp.float32),
    ] + [pltpu.SemaphoreType.DMA] * (3 * HNB),
)
def _hop_kernel(u_hbm, row_hbm, col_hbm, zeros_hbm, out_hbm,
                idx_c, *bufs):
    idxb = bufs[:HNB]
    rows = bufs[HNB:2 * HNB]
    acc = bufs[2 * HNB]
    semi = bufs[2 * HNB + 1:2 * HNB + 1 + HNB]
    semg = bufs[2 * HNB + 1 + HNB:2 * HNB + 1 + 2 * HNB]
    sems_ = bufs[2 * HNB + 1 + 2 * HNB:]
    c = lax.axis_index("c")
    s = lax.axis_index("s")
    wid = c * NS + s
    pltpu.sync_copy(zeros_hbm, acc.at[pl.ds(s * RPT, RPT)])
    pltpu.sync_copy(col_hbm.at[wid], idx_c)
    for b in range(HNB):
        pltpu.async_copy(row_hbm.at[wid, b], idxb[b], semi[b])
    plsc.subcore_barrier()

    def body(g, carry):
        for b in range(HNB):
            i = g * HNB + b

            @pl.when(g > 0)
            def _():
                pltpu.make_async_copy(
                    rows[b], acc.at[idx_c.at[0]], sems_[b]).wait()

            pltpu.make_async_copy(row_hbm.at[wid, 0], idxb[b], semi[b]).wait()
            pltpu.async_copy(u_hbm.at[idxb[b]], rows[b], semg[b])
        for b in range(HNB):
            i = g * HNB + b
            pltpu.make_async_copy(
                u_hbm.at[idxb[b]], rows[b], semg[b]).wait()

            @pl.when(i + HNB < HNCH)
            def _():
                pltpu.async_copy(row_hbm.at[wid, i + HNB], idxb[b], semi[b])

            pltpu.async_copy(rows[b], acc.at[idx_c.at[i]], sems_[b],
                             add=True)
        return carry

    lax.fori_loop(0, HNG, body, 0)
    for b in range(HREM):
        i = HNG * HNB + b
        pltpu.make_async_copy(rows[b], acc.at[idx_c.at[0]], sems_[b]).wait()
        pltpu.make_async_copy(row_hbm.at[wid, 0], idxb[b], semi[b]).wait()
        pltpu.async_copy(u_hbm.at[idxb[b]], rows[b], semg[b])
    for b in range(HREM):
        i = HNG * HNB + b
        pltpu.make_async_copy(u_hbm.at[idxb[b]], rows[b], semg[b]).wait()
        pltpu.async_copy(rows[b], acc.at[idx_c.at[i]], sems_[b], add=True)
    for b in range(HNB):
        pltpu.make_async_copy(rows[b], acc.at[idx_c.at[0]], sems_[b]).wait()
    plsc.subcore_barrier()
    pltpu.sync_copy(acc.at[pl.ds(s * RPT, RPT)],
                    out_hbm.at[c, pl.ds(s * RPT, RPT)])


_R = 2000
_GRID = N // _R


def _deg_of(d0, d1):
    return d0[:, :1] + d1[:, :1] + 1.0


def _mm_body(x_ref, wt_ref, d0_ref, d1_ref, y_ref):
    dinv = lax.rsqrt(_deg_of(d0_ref[...], d1_ref[...]))
    y_ref[...] = dinv * jnp.dot(x_ref[...], wt_ref[...],
                                preferred_element_type=jnp.float32)


def _comb_body(p0_ref, p1_ref, u_ref, d0_ref, d1_ref, o_ref):
    deg = _deg_of(d0_ref[...], d1_ref[...])
    o_ref[...] = (p0_ref[...] + p1_ref[...] + u_ref[...]) / deg


def _final_body(p0_ref, p1_ref, u_ref, d0_ref, d1_ref, b_ref, o_ref):
    dinv = lax.rsqrt(_deg_of(d0_ref[...], d1_ref[...]))
    o_ref[...] = (p0_ref[...] + p1_ref[...] + u_ref[...]) * dinv + b_ref[...]


_row_spec = pl.BlockSpec((_R, D), lambda i: (i, 0))
_w_spec = pl.BlockSpec((D, D), lambda i: (0, 0))
_b_spec = pl.BlockSpec((1, D), lambda i: (0, 0))
_out_row = jax.ShapeDtypeStruct((N, D), jnp.float32)

_mm_call = pl.pallas_call(
    _mm_body, grid=(_GRID,),
    in_specs=[_row_spec, _w_spec, _row_spec, _row_spec],
    out_specs=_row_spec, out_shape=_out_row)

_comb_call = pl.pallas_call(
    _comb_body, grid=(_GRID,),
    in_specs=[_row_spec, _row_spec, _row_spec, _row_spec, _row_spec],
    out_specs=_row_spec, out_shape=_out_row)

_final_call = pl.pallas_call(
    _final_body, grid=(_GRID,),
    in_specs=[_row_spec, _row_spec, _row_spec, _row_spec, _row_spec, _b_spec],
    out_specs=_row_spec, out_shape=_out_row)


def kernel(x, edge_index, W, b):
    npad = EPAD - E
    ar = jnp.arange(npad, dtype=jnp.int32)
    row = jnp.concatenate([edge_index[0], ar % N])
    col = jnp.concatenate([edge_index[1], N + ar % (NPAD - N)])
    row_h = row.reshape(NW, HNCH, HCH)
    col_h = col.reshape(NW, HNCH, HCH)
    col = col.reshape(NW, NCHUNK, CHUNK)
    wt = W.T
    ones_deg = jnp.ones((CHUNK, D), jnp.float32)
    zeros_row = jnp.zeros((RPT, D), jnp.float32)
    b2 = b.reshape(1, D)

    degp = _deg_kernel(col, ones_deg, zeros_row)
    d0, d1 = degp[0], degp[1]

    u0 = _mm_call(x, wt, d0, d1)
    p = _hop_kernel(u0, row_h, col_h, zeros_row)
    u1 = _comb_call(p[0], p[1], u0, d0, d1)
    p = _hop_kernel(u1, row_h, col_h, zeros_row)
    out = _final_call(p[0], p[1], u1, d0, d1, b2)
    return (out, out)


# speedup vs baseline: 3.2078x; 1.0518x over previous
import functools

import jax
import jax.numpy as jnp
from jax import lax
from jax.experimental import pallas as pl
from jax.experimental.pallas import tpu as pltpu
from jax.experimental.pallas import tpu_sc as plsc

N = 10000
E = 320000
D = 128
NC = 2
NS = 16
NW = NC * NS
EPW = 10240
EPAD = EPW * NW
CHUNK = 128
NCHUNK = EPW // CHUNK
NBUF = 2
NG = NCHUNK // NBUF
HCH = 80
HNCH = EPW // HCH
HNB = 4
HNG = HNCH // HNB
NPAD = 10240
RPT = NPAD // NS

_sc_mesh = plsc.VectorSubcoreMesh(core_axis_name="c", subcore_axis_name="s")


@functools.partial(
    pl.kernel,
    out_type=jax.ShapeDtypeStruct((NC, NPAD, D), jnp.float32),
    mesh=_sc_mesh,
    scratch_types=[
        pltpu.VMEM((NCHUNK, CHUNK), jnp.int32),
        pltpu.VMEM((CHUNK, D), jnp.float32),
        pltpu.VMEM_SHARED((NPAD, D), jnp.float32),
    ] + [pltpu.SemaphoreType.DMA] * NBUF,
)
def _deg_kernel(col_hbm, ones_hbm, zeros_hbm, out_hbm, idx_c, ones_v, tab,
                *sems):
    c = lax.axis_index("c")
    s = lax.axis_index("s")
    wid = c * NS + s
    pltpu.sync_copy(zeros_hbm, tab.at[pl.ds(s * RPT, RPT)])
    pltpu.sync_copy(col_hbm.at[wid], idx_c)
    pltpu.sync_copy(ones_hbm, ones_v)
    plsc.subcore_barrier()

    def body(g, carry):
        for b in range(NBUF):
            i = g * NBUF + b

            @pl.when(g > 0)
            def _():
                pltpu.make_async_copy(
                    ones_v, tab.at[idx_c.at[0]], sems[b]).wait()

            pltpu.async_copy(ones_v, tab.at[idx_c.at[i]], sems[b], add=True)
        return carry

    lax.fori_loop(0, NG, body, 0)
    for b in range(NBUF):
        pltpu.make_async_copy(ones_v, tab.at[idx_c.at[0]], sems[b]).wait()
    plsc.subcore_barrier()
    pltpu.sync_copy(tab.at[pl.ds(s * RPT, RPT)],
                    out_hbm.at[c, pl.ds(s * RPT, RPT)])


@functools.partial(
    pl.kernel,
    out_type=jax.ShapeDtypeStruct((NC, NPAD, D), jnp.float32),
    mesh=_sc_mesh,
    scratch_types=[pltpu.VMEM((HCH,), jnp.int32)] * HNB
      + [pltpu.VMEM((1, HCH), jnp.int32)] * HNB
      + [pltpu.VMEM((HCH, D), jnp.float32)] * HNB + [
        pltpu.VMEM_SHARED((NPAD, D), jnp.float32),
    ] + [pltpu.SemaphoreType.DMA] * (4 * HNB),
)
def _hop_kernel(u_hbm, row_hbm, col_hbm, zeros_hbm, out_hbm, *bufs):
    idxb = bufs[:HNB]
    dstb = bufs[HNB:2 * HNB]
    rows = bufs[2 * HNB:3 * HNB]
    acc = bufs[3 * HNB]
    semi = bufs[3 * HNB + 1:3 * HNB + 1 + HNB]
    semd = bufs[3 * HNB + 1 + HNB:3 * HNB + 1 + 2 * HNB]
    semg = bufs[3 * HNB + 1 + 2 * HNB:3 * HNB + 1 + 3 * HNB]
    sems_ = bufs[3 * HNB + 1 + 3 * HNB:]
    c = lax.axis_index("c")
    s = lax.axis_index("s")
    wid = c * NS + s
    pltpu.sync_copy(zeros_hbm, acc.at[pl.ds(s * RPT, RPT)])
    for b in range(HNB):
        pltpu.async_copy(row_hbm.at[wid, b], idxb[b], semi[b])
        pltpu.async_copy(col_hbm.at[wid, b], dstb[b], semd[b])
    plsc.subcore_barrier()

    def body(g, carry):
        for b in range(HNB):
            i = g * HNB + b

            @pl.when(g > 0)
            def _():
                pltpu.make_async_copy(
                    rows[b], acc.at[dstb[b].at[0]], sems_[b]).wait()
                pltpu.async_copy(col_hbm.at[wid, i], dstb[b], semd[b])

            pltpu.make_async_copy(row_hbm.at[wid, 0], idxb[b], semi[b]).wait()
            pltpu.async_copy(u_hbm.at[idxb[b]], rows[b], semg[b])
        for b in range(HNB):
            i = g * HNB + b
            pltpu.make_async_copy(
                u_hbm.at[idxb[b]], rows[b], semg[b]).wait()

            @pl.when(i + HNB < HNCH)
            def _():
                pltpu.async_copy(row_hbm.at[wid, i + HNB], idxb[b], semi[b])

            pltpu.make_async_copy(col_hbm.at[wid, 0], dstb[b], semd[b]).wait()
            pltpu.async_copy(rows[b], acc.at[dstb[b].at[0]], sems_[b],
                             add=True)
        return carry

    lax.fori_loop(0, HNG, body, 0)
    for b in range(HNB):
        pltpu.make_async_copy(rows[b], acc.at[dstb[b].at[0]], sems_[b]).wait()
    plsc.subcore_barrier()
    pltpu.sync_copy(acc.at[pl.ds(s * RPT, RPT)],
                    out_hbm.at[c, pl.ds(s * RPT, RPT)])


_R = 2000
_GRID = N // _R


def _deg_of(d0, d1):
    return d0[:, :1] + d1[:, :1] + 1.0


def _mm_body(x_ref, wt_ref, d0_ref, d1_ref, y_ref):
    dinv = lax.rsqrt(_deg_of(d0_ref[...], d1_ref[...]))
    y_ref[...] = dinv * jnp.dot(x_ref[...], wt_ref[...],
                                preferred_element_type=jnp.float32)


def _comb_body(p0_ref, p1_ref, u_ref, d0_ref, d1_ref, o_ref):
    deg = _deg_of(d0_ref[...], d1_ref[...])
    o_ref[...] = (p0_ref[...] + p1_ref[...] + u_ref[...]) / deg


def _final_body(p0_ref, p1_ref, u_ref, d0_ref, d1_ref, b_ref, o_ref):
    dinv = lax.rsqrt(_deg_of(d0_ref[...], d1_ref[...]))
    o_ref[...] = (p0_ref[...] + p1_ref[...] + u_ref[...]) * dinv + b_ref[...]


_row_spec = pl.BlockSpec((_R, D), lambda i: (i, 0))
_w_spec = pl.BlockSpec((D, D), lambda i: (0, 0))
_b_spec = pl.BlockSpec((1, D), lambda i: (0, 0))
_out_row = jax.ShapeDtypeStruct((N, D), jnp.float32)

_mm_call = pl.pallas_call(
    _mm_body, grid=(_GRID,),
    in_specs=[_row_spec, _w_spec, _row_spec, _row_spec],
    out_specs=_row_spec, out_shape=_out_row)

_comb_call = pl.pallas_call(
    _comb_body, grid=(_GRID,),
    in_specs=[_row_spec, _row_spec, _row_spec, _row_spec, _row_spec],
    out_specs=_row_spec, out_shape=_out_row)

_final_call = pl.pallas_call(
    _final_body, grid=(_GRID,),
    in_specs=[_row_spec, _row_spec, _row_spec, _row_spec, _row_spec, _b_spec],
    out_specs=_row_spec, out_shape=_out_row)


def kernel(x, edge_index, W, b):
    npad = EPAD - E
    ar = jnp.arange(npad, dtype=jnp.int32)
    row = jnp.concatenate([edge_index[0], ar % N])
    col = jnp.concatenate([edge_index[1], N + ar % (NPAD - N)])
    row_h = row.reshape(NW, HNCH, HCH)
    col_h = col.reshape(NW, HNCH, 1, HCH)
    col = col.reshape(NW, NCHUNK, CHUNK)
    wt = W.T
    ones_deg = jnp.ones((CHUNK, D), jnp.float32)
    zeros_row = jnp.zeros((RPT, D), jnp.float32)
    b2 = b.reshape(1, D)

    degp = _deg_kernel(col, ones_deg, zeros_row)
    d0, d1 = degp[0], degp[1]

    u0 = _mm_call(x, wt, d0, d1)
    p = _hop_kernel(u0, row_h, col_h, zeros_row)
    u1 = _comb_call(p[0], p[1], u0, d0, d1)
    p = _hop_kernel(u1, row_h, col_h, zeros_row)
    out = _final_call(p[0], p[1], u1, d0, d1, b2)
    return (out, out)


# speedup vs baseline: 3.2105x; 1.0008x over previous
import functools

import jax
import jax.numpy as jnp
from jax import lax
from jax.experimental import pallas as pl
from jax.experimental.pallas import tpu as pltpu
from jax.experimental.pallas import tpu_sc as plsc

N = 10000
E = 320000
D = 128
NC = 2
NS = 16
NW = NC * NS
EPW = 10240
EPAD = EPW * NW
CHUNK = 128
NCHUNK = EPW // CHUNK
NBUF = 2
NG = NCHUNK // NBUF
HCH = 80
HNCH = EPW // HCH
HNB = 4
HNG = HNCH // HNB
NPAD = 10240
RPT = NPAD // NS

_sc_mesh = plsc.VectorSubcoreMesh(core_axis_name="c", subcore_axis_name="s")


DNB = 4


@functools.partial(
    pl.kernel,
    out_type=jax.ShapeDtypeStruct((NC, NPAD, D), jnp.float32),
    mesh=_sc_mesh,
    scratch_types=[
        pltpu.VMEM((NCHUNK, CHUNK), jnp.int32),
        pltpu.VMEM((CHUNK, D), jnp.float32),
        pltpu.VMEM_SHARED((NPAD, D), jnp.float32),
    ] + [pltpu.SemaphoreType.DMA] * DNB,
)
def _deg_kernel(col_hbm, ones_hbm, zeros_hbm, out_hbm, idx_c, ones_v, tab,
                *sems):
    c = lax.axis_index("c")
    s = lax.axis_index("s")
    wid = c * NS + s
    pltpu.sync_copy(zeros_hbm, tab.at[pl.ds(s * RPT, RPT)])
    pltpu.sync_copy(col_hbm.at[wid], idx_c)
    pltpu.sync_copy(ones_hbm, ones_v)
    plsc.subcore_barrier()

    def body(g, carry):
        for b in range(DNB):
            i = g * DNB + b

            @pl.when(g > 0)
            def _():
                pltpu.make_async_copy(
                    ones_v, tab.at[idx_c.at[0]], sems[b]).wait()

            pltpu.async_copy(ones_v, tab.at[idx_c.at[i]], sems[b], add=True)
        return carry

    lax.fori_loop(0, NCHUNK // DNB, body, 0)
    for b in range(DNB):
        pltpu.make_async_copy(ones_v, tab.at[idx_c.at[0]], sems[b]).wait()
    plsc.subcore_barrier()
    pltpu.sync_copy(tab.at[pl.ds(s * RPT, RPT)],
                    out_hbm.at[c, pl.ds(s * RPT, RPT)])


@functools.partial(
    pl.kernel,
    out_type=jax.ShapeDtypeStruct((NC, NPAD, D), jnp.float32),
    mesh=_sc_mesh,
    scratch_types=[pltpu.VMEM((HCH,), jnp.int32)] * HNB
      + [pltpu.VMEM((1, HCH), jnp.int32)] * HNB
      + [pltpu.VMEM((HCH, D), jnp.float32)] * HNB + [
        pltpu.VMEM_SHARED((NPAD, D), jnp.float32),
    ] + [pltpu.SemaphoreType.DMA] * (4 * HNB),
)
def _hop_kernel(u_hbm, row_hbm, col_hbm, zeros_hbm, out_hbm, *bufs):
    idxb = bufs[:HNB]
    dstb = bufs[HNB:2 * HNB]
    rows = bufs[2 * HNB:3 * HNB]
    acc = bufs[3 * HNB]
    semi = bufs[3 * HNB + 1:3 * HNB + 1 + HNB]
    semd = bufs[3 * HNB + 1 + HNB:3 * HNB + 1 + 2 * HNB]
    semg = bufs[3 * HNB + 1 + 2 * HNB:3 * HNB + 1 + 3 * HNB]
    sems_ = bufs[3 * HNB + 1 + 3 * HNB:]
    c = lax.axis_index("c")
    s = lax.axis_index("s")
    wid = c * NS + s
    pltpu.sync_copy(zeros_hbm, acc.at[pl.ds(s * RPT, RPT)])
    for b in range(HNB):
        pltpu.async_copy(row_hbm.at[wid, b], idxb[b], semi[b])
        pltpu.async_copy(col_hbm.at[wid, b], dstb[b], semd[b])
    plsc.subcore_barrier()

    def body(g, carry):
        for b in range(HNB):
            i = g * HNB + b

            @pl.when(g > 0)
            def _():
                pltpu.make_async_copy(
                    rows[b], acc.at[dstb[b].at[0]], sems_[b]).wait()
                pltpu.async_copy(col_hbm.at[wid, i], dstb[b], semd[b])

            pltpu.make_async_copy(row_hbm.at[wid, 0], idxb[b], semi[b]).wait()
            pltpu.async_copy(u_hbm.at[idxb[b]], rows[b], semg[b])
        for b in range(HNB):
            i = g * HNB + b
            pltpu.make_async_copy(
                u_hbm.at[idxb[b]], rows[b], semg[b]).wait()

            @pl.when(i + HNB < HNCH)
            def _():
                pltpu.async_copy(row_hbm.at[wid, i + HNB], idxb[b], semi[b])

            pltpu.make_async_copy(col_hbm.at[wid, 0], dstb[b], semd[b]).wait()
            pltpu.async_copy(rows[b], acc.at[dstb[b].at[0]], sems_[b],
                             add=True)
        return carry

    lax.fori_loop(0, HNG, body, 0)
    for b in range(HNB):
        pltpu.make_async_copy(rows[b], acc.at[dstb[b].at[0]], sems_[b]).wait()
    plsc.subcore_barrier()
    pltpu.sync_copy(acc.at[pl.ds(s * RPT, RPT)],
                    out_hbm.at[c, pl.ds(s * RPT, RPT)])


_R = 2000
_GRID = N // _R


def _deg_of(d0, d1):
    return d0[:, :1] + d1[:, :1] + 1.0


def _mm_body(x_ref, wt_ref, d0_ref, d1_ref, y_ref):
    dinv = lax.rsqrt(_deg_of(d0_ref[...], d1_ref[...]))
    y_ref[...] = dinv * jnp.dot(x_ref[...], wt_ref[...],
                                preferred_element_type=jnp.float32)


def _comb_body(p0_ref, p1_ref, u_ref, d0_ref, d1_ref, o_ref):
    deg = _deg_of(d0_ref[...], d1_ref[...])
    o_ref[...] = (p0_ref[...] + p1_ref[...] + u_ref[...]) / deg


def _final_body(p0_ref, p1_ref, u_ref, d0_ref, d1_ref, b_ref, o_ref):
    dinv = lax.rsqrt(_deg_of(d0_ref[...], d1_ref[...]))
    o_ref[...] = (p0_ref[...] + p1_ref[...] + u_ref[...]) * dinv + b_ref[...]


_row_spec = pl.BlockSpec((_R, D), lambda i: (i, 0))
_w_spec = pl.BlockSpec((D, D), lambda i: (0, 0))
_b_spec = pl.BlockSpec((1, D), lambda i: (0, 0))
_out_row = jax.ShapeDtypeStruct((N, D), jnp.float32)

_mm_call = pl.pallas_call(
    _mm_body, grid=(_GRID,),
    in_specs=[_row_spec, _w_spec, _row_spec, _row_spec],
    out_specs=_row_spec, out_shape=_out_row)

_comb_call = pl.pallas_call(
    _comb_body, grid=(_GRID,),
    in_specs=[_row_spec, _row_spec, _row_spec, _row_spec, _row_spec],
    out_specs=_row_spec, out_shape=_out_row)

_final_call = pl.pallas_call(
    _final_body, grid=(_GRID,),
    in_specs=[_row_spec, _row_spec, _row_spec, _row_spec, _row_spec, _b_spec],
    out_specs=_row_spec, out_shape=_out_row)


def kernel(x, edge_index, W, b):
    npad = EPAD - E
    ar = jnp.arange(npad, dtype=jnp.int32)
    row = jnp.concatenate([edge_index[0], ar % N])
    col = jnp.concatenate([edge_index[1], N + ar % (NPAD - N)])
    row_h = row.reshape(NW, HNCH, HCH)
    col_h = col.reshape(NW, HNCH, 1, HCH)
    col = col.reshape(NW, NCHUNK, CHUNK)
    wt = W.T
    ones_deg = jnp.ones((CHUNK, D), jnp.float32)
    zeros_row = jnp.zeros((RPT, D), jnp.float32)
    b2 = b.reshape(1, D)

    degp = _deg_kernel(col, ones_deg, zeros_row)
    d0, d1 = degp[0], degp[1]

    u0 = _mm_call(x, wt, d0, d1)
    p = _hop_kernel(u0, row_h, col_h, zeros_row)
    u1 = _comb_call(p[0], p[1], u0, d0, d1)
    p = _hop_kernel(u1, row_h, col_h, zeros_row)
    out = _final_call(p[0], p[1], u1, d0, d1, b2)
    return (out, out)
